# Initial kernel scaffold; baseline (speedup 1.0000x reference)
import jax
import jax.numpy as jnp
from jax.experimental import pallas as pl


def kernel(x, weights, coords):
    raise NotImplementedError("write your pallas kernel here")


# speedup vs baseline: 33.4272x; 33.4272x over previous
import functools

import jax
import jax.numpy as jnp
from jax import lax
from jax.experimental import pallas as pl
from jax.experimental.pallas import tpu as pltpu
from jax.experimental.pallas import tpu_sc as plsc

NC = 2
NS = 16
LANES = 16

CH = 128
K = 8
MAC = CH * K


def _sc_spmm(xT, rows2, cols2, w32, zeros, out_features):
    nnz = w32.shape[0]
    n_tiles = NC * NS
    pt = nnz // n_tiles
    blocks_pt = pt // CH
    n_mac = pt // MAC
    rpt = out_features // NS

    mesh = plsc.VectorSubcoreMesh(core_axis_name="c", subcore_axis_name="s")

    @functools.partial(
        pl.kernel,
        out_type=jax.ShapeDtypeStruct((NC, out_features, LANES), jnp.float32),
        mesh=mesh,
        scratch_types=[
            pltpu.VMEM((K, CH), jnp.int32),
            pltpu.VMEM((K, CH), jnp.int32),
            pltpu.VMEM((MAC,), jnp.float32),
            pltpu.VMEM((MAC, LANES), jnp.float32),
            pltpu.VMEM_SHARED((out_features, LANES), jnp.float32),
            pltpu.SemaphoreType.DMA,
            pltpu.SemaphoreType.DMA,
        ],
        compiler_params=pltpu.CompilerParams(use_tc_tiling_on_sc=False),
    )
    def spmm(xT_hbm, rows_hbm, cols_hbm, w_hbm, z_hbm, out_hbm,
             rows_v, cols_v, w_v, g_v, acc, gsem, ssem):
        core = lax.axis_index("c")
        sub = lax.axis_index("s")
        wid = sub * NC + core

        pltpu.sync_copy(z_hbm.at[pl.ds(sub * rpt, rpt)],
                        acc.at[pl.ds(sub * rpt, rpt)])
        plsc.subcore_barrier()

        blk0 = wid * blocks_pt

        def macro(m, carry):
            b = blk0 + m * K
            pltpu.sync_copy(cols_hbm.at[pl.ds(b, K)], cols_v)
            pltpu.sync_copy(rows_hbm.at[pl.ds(b, K)], rows_v)
            pltpu.sync_copy(w_hbm.at[pl.ds(b * CH, MAC)], w_v)
            gathers = [
                pltpu.async_copy(xT_hbm.at[cols_v.at[j]],
                                 g_v.at[pl.ds(j * CH, CH)], gsem)
                for j in range(K)
            ]
            for g in gathers:
                g.wait()

            def compute(t, c):
                base = t * LANES
                w16 = w_v[pl.ds(base, LANES)]
                for l in range(LANES):
                    g_v[base + l] = g_v[base + l] * w16[l]
                return c

            lax.fori_loop(0, MAC // LANES, compute, 0, unroll=2)

            scatters = [
                pltpu.async_copy(g_v.at[pl.ds(j * CH, CH)],
                                 acc.at[rows_v.at[j]], ssem, add=True)
                for j in range(K)
            ]
            for s in scatters:
                s.wait()
            return carry

        lax.fori_loop(0, n_mac, macro, 0)

        plsc.subcore_barrier()
        pltpu.sync_copy(acc.at[pl.ds(sub * rpt, rpt)],
                        out_hbm.at[core, pl.ds(sub * rpt, rpt)])

    return spmm(xT, rows2, cols2, w32, zeros)


def _combine(parts, out_features):
    cb = 2048

    def body(p_ref, o_ref):
        o_ref[...] = p_ref[0] + p_ref[1]

    return pl.pallas_call(
        body,
        grid=(out_features // cb,),
        in_specs=[pl.BlockSpec((NC, cb, LANES), lambda i: (0, i, 0))],
        out_specs=pl.BlockSpec((cb, LANES), lambda i: (i, 0)),
        out_shape=jax.ShapeDtypeStruct((out_features, LANES), jnp.float32),
    )(parts)


def kernel(x, weights, coords):
    batch, in_features = x.shape
    out_features = in_features
    nnz = weights.shape[0]

    rows2 = coords[:, 0].reshape(nnz // CH, CH)
    cols2 = coords[:, 1].reshape(nnz // CH, CH)
    w32 = weights.astype(jnp.float32)
    xT = x.astype(jnp.float32).T
    zeros = jnp.zeros((out_features, LANES), jnp.float32)

    parts = _sc_spmm(xT, rows2, cols2, w32, zeros, out_features)
    outT = _combine(parts, out_features)
    return outT.T.astype(x.dtype)


# speedup vs baseline: 46.2870x; 1.3847x over previous
import functools

import jax
import jax.numpy as jnp
from jax import lax
from jax.experimental import pallas as pl
from jax.experimental.pallas import tpu as pltpu
from jax.experimental.pallas import tpu_sc as plsc

NC = 2
NS = 16
LANES = 16

CH = 128
K = 8
MAC = CH * K
ZR = 128


def _sc_spmm(xT, rows2, cols2, w32, out_features):
    nnz = w32.shape[0]
    n_tiles = NC * NS
    pt = nnz // n_tiles
    blocks_pt = pt // CH
    n_mac = pt // MAC
    rpt = out_features // NS

    mesh = plsc.VectorSubcoreMesh(core_axis_name="c", subcore_axis_name="s")

    @functools.partial(
        pl.kernel,
        out_type=jax.ShapeDtypeStruct((NC, out_features, LANES), jnp.float32),
        mesh=mesh,
        scratch_types=[
            pltpu.VMEM((2, K, CH), jnp.int32),
            pltpu.VMEM((2, K, CH), jnp.int32),
            pltpu.VMEM((2, MAC), jnp.float32),
            pltpu.VMEM((2, MAC, LANES), jnp.float32),
            pltpu.VMEM((ZR, LANES), jnp.float32),
            pltpu.VMEM_SHARED((out_features, LANES), jnp.float32),
            pltpu.SemaphoreType.DMA,
            pltpu.SemaphoreType.DMA,
        ],
        compiler_params=pltpu.CompilerParams(use_tc_tiling_on_sc=False),
    )
    def spmm(xT_hbm, rows_hbm, cols_hbm, w_hbm, out_hbm,
             rows_v, cols_v, w_v, g_v, z_v, acc, gsem, ssem):
        core = lax.axis_index("c")
        sub = lax.axis_index("s")
        wid = sub * NC + core

        def zfill(i, c):
            z_v[i] = jnp.zeros((LANES,), jnp.float32)
            return c

        lax.fori_loop(0, ZR, zfill, 0, unroll=8)
        for q in range(rpt // ZR):
            pltpu.sync_copy(z_v, acc.at[pl.ds(sub * rpt + q * ZR, ZR)])
        plsc.subcore_barrier()

        blk0 = wid * blocks_pt

        def fire_chunk(s, b):
            pltpu.sync_copy(cols_hbm.at[pl.ds(b, K)], cols_v.at[s])
            pltpu.sync_copy(rows_hbm.at[pl.ds(b, K)], rows_v.at[s])
            pltpu.sync_copy(w_hbm.at[pl.ds(b * CH, MAC)], w_v.at[s])
            return [
                pltpu.async_copy(xT_hbm.at[cols_v.at[s, j]],
                                 g_v.at[s, pl.ds(j * CH, CH)], gsem)
                for j in range(K)
            ]

        def drain_scatters(s):
            for j in range(K):
                pltpu.make_async_copy(
                    g_v.at[s, pl.ds(j * CH, CH)],
                    acc.at[rows_v.at[s, j]], ssem).wait()

        def compute_and_scatter(s, gathers):
            for g in gathers:
                g.wait()

            def compute(t, c):
                base = t * LANES
                w16 = w_v[s, pl.ds(base, LANES)]
                for l in range(LANES):
                    g_v[s, base + l] = g_v[s, base + l] * w16[l]
                return c

            lax.fori_loop(0, MAC // LANES, compute, 0, unroll=2)
            for j in range(K):
                pltpu.async_copy(g_v.at[s, pl.ds(j * CH, CH)],
                                 acc.at[rows_v.at[s, j]], ssem, add=True)

        def macro(t, carry):
            b0 = blk0 + 2 * t * K

            @pl.when(t > 0)
            def _():
                drain_scatters(0)

            gath0 = fire_chunk(0, b0)

            @pl.when(t > 0)
            def _():
                drain_scatters(1)

            gath1 = fire_chunk(1, b0 + K)
            compute_and_scatter(0, gath0)
            compute_and_scatter(1, gath1)
            return carry

        lax.fori_loop(0, n_mac // 2, macro, 0)
        drain_scatters(0)
        drain_scatters(1)

        plsc.subcore_barrier()
        pltpu.sync_copy(acc.at[pl.ds(sub * rpt, rpt)],
                        out_hbm.at[core, pl.ds(sub * rpt, rpt)])

    return spmm(xT, rows2, cols2, w32)


def _combine(parts, out_features):
    cb = 2048

    def body(p_ref, o_ref):
        o_ref[...] = p_ref[0] + p_ref[1]

    return pl.pallas_call(
        body,
        grid=(out_features // cb,),
        in_specs=[pl.BlockSpec((NC, cb, LANES), lambda i: (0, i, 0))],
        out_specs=pl.BlockSpec((cb, LANES), lambda i: (i, 0)),
        out_shape=jax.ShapeDtypeStruct((out_features, LANES), jnp.float32),
    )(parts)


def kernel(x, weights, coords):
    batch, in_features = x.shape
    out_features = in_features
    nnz = weights.shape[0]

    rows2 = coords[:, 0].reshape(nnz // CH, CH)
    cols2 = coords[:, 1].reshape(nnz // CH, CH)
    w32 = weights.astype(jnp.float32)
    xT = x.astype(jnp.float32).T

    parts = _sc_spmm(xT, rows2, cols2, w32, out_features)
    outT = _combine(parts, out_features)
    return outT.T.astype(x.dtype)


# speedup vs baseline: 46.4323x; 1.0031x over previous
import functools

import jax
import jax.numpy as jnp
from jax import lax
from jax.experimental import pallas as pl
from jax.experimental.pallas import tpu as pltpu
from jax.experimental.pallas import tpu_sc as plsc

NC = 2
NS = 16
LANES = 16

CH = 128
K = 8
MAC = CH * K
ZR = 128


def _sc_spmm(xT, rows3, cols3, w32, out_features):
    nnz = w32.shape[0]
    n_tiles = NC * NS
    pt = nnz // n_tiles
    n_mac = pt // MAC
    rpt = out_features // NS

    mesh = plsc.VectorSubcoreMesh(core_axis_name="c", subcore_axis_name="s")

    @functools.partial(
        pl.kernel,
        out_type=jax.ShapeDtypeStruct((NC, out_features, LANES), jnp.float32),
        mesh=mesh,
        scratch_types=[
            pltpu.VMEM((MAC,), jnp.int32),
            pltpu.VMEM((MAC,), jnp.int32),
            pltpu.VMEM((MAC,), jnp.int32),
            pltpu.VMEM((MAC,), jnp.int32),
            pltpu.VMEM((2, MAC), jnp.float32),
            pltpu.VMEM((MAC, LANES), jnp.float32),
            pltpu.VMEM((MAC, LANES), jnp.float32),
            pltpu.VMEM((ZR, LANES), jnp.float32),
            pltpu.VMEM_SHARED((out_features, LANES), jnp.float32),
            pltpu.SemaphoreType.DMA,
            pltpu.SemaphoreType.DMA,
        ],
        compiler_params=pltpu.CompilerParams(use_tc_tiling_on_sc=False),
    )
    def spmm(xT_hbm, rows_hbm, cols_hbm, w_hbm, out_hbm,
             rows_v0, rows_v1, cols_v0, cols_v1, w_v, g_v0, g_v1,
             z_v, acc, gsem, ssem):
        rows_b = (rows_v0, rows_v1)
        cols_b = (cols_v0, cols_v1)
        g_b = (g_v0, g_v1)
        core = lax.axis_index("c")
        sub = lax.axis_index("s")
        wid = sub * NC + core

        def zfill(i, c):
            z_v[i] = jnp.zeros((LANES,), jnp.float32)
            return c

        lax.fori_loop(0, ZR, zfill, 0, unroll=8)
        for q in range(rpt // ZR):
            pltpu.sync_copy(z_v, acc.at[pl.ds(sub * rpt + q * ZR, ZR)])
        plsc.subcore_barrier()

        mac0 = wid * n_mac

        def fire_chunk(s, m):
            pltpu.sync_copy(cols_hbm.at[pl.ds(m * MAC, MAC)], cols_b[s])
            pltpu.sync_copy(rows_hbm.at[pl.ds(m * MAC, MAC)], rows_b[s])
            pltpu.sync_copy(w_hbm.at[pl.ds(m * MAC, MAC)], w_v.at[s])
            return pltpu.async_copy(xT_hbm.at[cols_b[s]], g_b[s], gsem)

        def drain_scatter(s):
            pltpu.make_async_copy(g_b[s], acc.at[rows_b[s]], ssem).wait()

        def compute_and_scatter(s, gather):
            gather.wait()
            g_v = g_b[s]

            def compute(t, c):
                base = t * LANES
                w16 = w_v[s, pl.ds(base, LANES)]
                for l in range(LANES):
                    g_v[base + l] = g_v[base + l] * w16[l]
                return c

            lax.fori_loop(0, MAC // LANES, compute, 0, unroll=2)
            pltpu.async_copy(g_b[s], acc.at[rows_b[s]], ssem, add=True)

        def macro(t, carry):
            m0 = mac0 + 2 * t

            @pl.when(t > 0)
            def _():
                drain_scatter(0)

            gath0 = fire_chunk(0, m0)

            @pl.when(t > 0)
            def _():
                drain_scatter(1)

            gath1 = fire_chunk(1, m0 + 1)
            compute_and_scatter(0, gath0)
            compute_and_scatter(1, gath1)
            return carry

        lax.fori_loop(0, n_mac // 2, macro, 0)
        drain_scatter(0)
        drain_scatter(1)

        plsc.subcore_barrier()
        pltpu.sync_copy(acc.at[pl.ds(sub * rpt, rpt)],
                        out_hbm.at[core, pl.ds(sub * rpt, rpt)])

    return spmm(xT, rows3, cols3, w32)


def _combine(parts, out_features):
    cb = 2048

    def body(p_ref, o_ref):
        o_ref[...] = p_ref[0] + p_ref[1]

    return pl.pallas_call(
        body,
        grid=(out_features // cb,),
        in_specs=[pl.BlockSpec((NC, cb, LANES), lambda i: (0, i, 0))],
        out_specs=pl.BlockSpec((cb, LANES), lambda i: (i, 0)),
        out_shape=jax.ShapeDtypeStruct((out_features, LANES), jnp.float32),
    )(parts)


def kernel(x, weights, coords):
    batch, in_features = x.shape
    out_features = in_features
    nnz = weights.shape[0]

    rows3 = coords[:, 0]
    cols3 = coords[:, 1]
    w32 = weights.astype(jnp.float32)
    xT = x.astype(jnp.float32).T

    parts = _sc_spmm(xT, rows3, cols3, w32, out_features)
    outT = _combine(parts, out_features)
    return outT.T.astype(x.dtype)
